# baseline (speedup 1.0000x reference)
import functools

import ml_dtypes
import numpy as np
import jax
import jax.numpy as jnp
from jax import lax
from jax.experimental import pallas as pl
from jax.experimental.pallas import tpu as pltpu
from jax.experimental.pallas import tpu_sc as plsc

D = 768
NC = 2
NS = 16
NW = NC * NS
LANES = 16
CH = 16


@functools.lru_cache(maxsize=None)
def _pe_table_np(seq_len: int, d: int):
    pos = np.arange(seq_len, dtype=np.float64).reshape(-1, 1)
    i = np.arange(0, d, 2, dtype=np.float64).reshape(1, -1)
    denom = np.power(10000.0, i / d)
    pe = np.zeros((seq_len, d), dtype=np.float32)
    pe[:, 0::2] = np.sin(pos / denom)
    pe[:, 1::2] = np.cos(pos / denom)
    bf = pe.astype(ml_dtypes.bfloat16).view(np.uint16)
    v = bf.reshape(seq_len, d // 32, 32).astype(np.uint32)
    words = v[:, :, :16] | (v[:, :, 16:] << 16)
    return words.reshape(seq_len, d // 2).view(np.int32)


@functools.lru_cache(maxsize=None)
def _build(batch: int, seq_len: int, vocab: int, d: int):
    tok = batch * seq_len
    assert seq_len % NW == 0
    ppw = seq_len // NW
    assert ppw % CH == 0
    npb = ppw // CH
    nch = npb * batch
    assert npb % 2 == 0 and batch % 2 == 0

    mesh = plsc.VectorSubcoreMesh(
        core_axis_name="c", subcore_axis_name="s",
        num_cores=NC, num_subcores=NS,
    )

    @functools.partial(
        pl.kernel,
        out_type=jax.ShapeDtypeStruct((tok, d), jnp.float32),
        mesh=mesh,
        scratch_types=[
            pltpu.VMEM((nch, CH), jnp.int32),
            pltpu.VMEM((CH, d), jnp.float32),
            pltpu.VMEM((CH, d), jnp.float32),
            pltpu.VMEM((CH, d), jnp.float32),
            pltpu.VMEM((CH, d), jnp.float32),
            pltpu.VMEM((CH * d // 2,), jnp.int32),
            pltpu.VMEM((CH * d // 2,), jnp.int32),
            pltpu.SemaphoreType.DMA,
            pltpu.SemaphoreType.DMA,
            pltpu.SemaphoreType.DMA,
            pltpu.SemaphoreType.DMA,
            pltpu.SemaphoreType.DMA,
            pltpu.SemaphoreType.DMA,
        ],
    )
    def emb_kernel(ids_hbm, table_hbm, pe_hbm, out_hbm,
                   idx_all, in0, in1, out0, out1, pe0, pe1,
                   g0, g1, p0, p1, s0, s1):
        wid = lax.axis_index("s") * NC + lax.axis_index("c")
        pos0 = wid * ppw

        gbufs = ((in0, out0, g0, s0), (in1, out1, g1, s1))
        pebufs = ((pe0, p0), (pe1, p1))

        def gather_cp(c, inb, gs):
            return pltpu.make_async_copy(table_hbm.at[idx_all.at[c]], inb, gs)

        def pe_cp(p, peb, ps):
            return pltpu.make_async_copy(
                pe_hbm.at[pl.ds((pos0 + p * CH) * (d // 2), CH * d // 2)],
                peb, ps)

        def store_cp(c, outb, ss):
            row0 = lax.rem(c, batch) * seq_len + pos0 + lax.div(c, batch) * CH
            return pltpu.make_async_copy(
                outb, out_hbm.at[pl.ds(row0, CH)], ss)

        pltpu.sync_copy(ids_hbm.at[wid], idx_all)
        for par in range(2):
            inb, outb, gs, ss = gbufs[par]
            gather_cp(par, inb, gs).start()
        pe_cp(0, pe0, p0).start()

        def outer(i, carry):
            for pp in range(2):
                p = i * 2 + pp
                peb, ps = pebufs[pp]
                pe_cp(p, peb, ps).wait()

                @pl.when(p + 1 < npb)
                def _():
                    pe_cp(p + 1, pebufs[1 - pp][0], pebufs[1 - pp][1]).start()

                for b in range(batch):
                    c = p * batch + b
                    par = b % 2
                    inb, outb, gs, ss = gbufs[par]
                    gather_cp(c, inb, gs).wait()

                    @pl.when(c >= 2)
                    def _():
                        store_cp(c - 2, outb, ss).wait()

                    def add_row(r, rcarry):
                        for k in range(d // (2 * LANES)):
                            w = peb[pl.ds(r * (d // 2) + k * LANES, LANES)]
                            pa = lax.bitcast_convert_type(w << 16, jnp.float32)
                            pb = lax.bitcast_convert_type(
                                w & jnp.int32(-65536), jnp.float32)
                            sl0 = pl.ds(k * 2 * LANES, LANES)
                            sl1 = pl.ds(k * 2 * LANES + LANES, LANES)
                            outb[r, sl0] = inb[r, sl0] + pa
                            outb[r, sl1] = inb[r, sl1] + pb
                        return rcarry

                    lax.fori_loop(0, CH, add_row, 0)
                    store_cp(c, outb, ss).start()

                    @pl.when(c + 2 < nch)
                    def _():
                        gather_cp(c + 2, inb, gs).start()
            return carry

        lax.fori_loop(0, npb // 2, outer, 0)

        for par in range(2):
            inb, outb, gs, ss = gbufs[par]
            store_cp(nch - 2 + par, outb, ss).wait()

    return emb_kernel


def kernel(token_ids, table):
    b, s = token_ids.shape
    vocab, d = table.shape
    ids = token_ids.astype(jnp.int32).reshape(b, NW, -1, CH).transpose(1, 2, 0, 3)
    ids = ids.reshape(NW, -1, CH)
    pe = jnp.asarray(_pe_table_np(s, d)).reshape(-1)
    out = _build(b, s, vocab, d)(ids, table, pe)
    return out.reshape(b, s, d)


# speedup vs baseline: 1.3729x; 1.3729x over previous
import functools

import numpy as np
import jax
import jax.numpy as jnp
from jax import lax
from jax.experimental import pallas as pl
from jax.experimental.pallas import tpu as pltpu
from jax.experimental.pallas import tpu_sc as plsc

D = 768
NC = 2
NS = 16
NW = NC * NS
LANES = 16
CH = 8


@functools.lru_cache(maxsize=None)
def _pe_table_np(seq_len: int, d: int):
    pos = np.arange(seq_len, dtype=np.float64).reshape(-1, 1)
    i = np.arange(0, d, 2, dtype=np.float64).reshape(1, -1)
    denom = np.power(10000.0, i / d)
    pe = np.zeros((seq_len, d), dtype=np.float32)
    pe[:, 0::2] = np.sin(pos / denom)
    pe[:, 1::2] = np.cos(pos / denom)
    return pe


@functools.lru_cache(maxsize=None)
def _build(batch: int, seq_len: int, vocab: int, d: int):
    tok = batch * seq_len
    assert seq_len % NW == 0
    ppw = seq_len // NW
    assert ppw % CH == 0
    npb = ppw // CH
    rows = batch * CH
    assert npb % 2 == 0 and npb >= 4

    mesh = plsc.VectorSubcoreMesh(
        core_axis_name="c", subcore_axis_name="s",
        num_cores=NC, num_subcores=NS,
    )

    @functools.partial(
        pl.kernel,
        out_type=jax.ShapeDtypeStruct((tok, d), jnp.float32),
        mesh=mesh,
        scratch_types=[
            pltpu.VMEM((npb, rows), jnp.int32),
            pltpu.VMEM((rows, d), jnp.float32),
            pltpu.VMEM((rows, d), jnp.float32),
            pltpu.VMEM((rows, d), jnp.float32),
            pltpu.VMEM((rows, d), jnp.float32),
            pltpu.VMEM((CH, d), jnp.float32),
            pltpu.VMEM((CH, d), jnp.float32),
            pltpu.SemaphoreType.DMA,
            pltpu.SemaphoreType.DMA,
            pltpu.SemaphoreType.DMA,
            pltpu.SemaphoreType.DMA,
            pltpu.SemaphoreType.DMA,
            pltpu.SemaphoreType.DMA,
        ],
    )
    def emb_kernel(ids_hbm, table_hbm, pe_hbm, out_hbm,
                   idx_all, in0, in1, out0, out1, pe0, pe1,
                   g0, g1, p0, p1, s0, s1):
        wid = lax.axis_index("s") * NC + lax.axis_index("c")
        pos0 = wid * ppw

        bufs = ((in0, out0, pe0, g0, p0, s0), (in1, out1, pe1, g1, p1, s1))

        def gather_cp(p, inb, gs):
            return pltpu.make_async_copy(table_hbm.at[idx_all.at[p]], inb, gs)

        def pe_cp(p, peb, ps):
            return pltpu.make_async_copy(
                pe_hbm.at[pl.ds(pos0 + p * CH, CH)], peb, ps)

        def store_cp(p, bat, outb, ss):
            row0 = bat * seq_len + pos0 + p * CH
            return pltpu.make_async_copy(
                outb.at[pl.ds(bat * CH, CH)],
                out_hbm.at[pl.ds(row0, CH)], ss)

        pltpu.sync_copy(ids_hbm.at[wid], idx_all)
        for par in range(2):
            inb, outb, peb, gs, ps, ss = bufs[par]
            gather_cp(par, inb, gs).start()
            pe_cp(par, peb, ps).start()

        def iter2(i, carry):
            for par in range(2):
                p = i * 2 + par
                inb, outb, peb, gs, ps, ss = bufs[par]
                gather_cp(p, inb, gs).wait()
                pe_cp(p, peb, ps).wait()

                @pl.when(p >= 2)
                def _():
                    for bat in range(batch):
                        store_cp(p - 2, bat, outb, ss).wait()

                def add_row(r, rcarry):
                    for k in range(d // LANES):
                        sl = pl.ds(k * LANES, LANES)
                        pv = peb[r, sl]
                        for bat in range(batch):
                            outb[bat * CH + r, sl] = inb[bat * CH + r, sl] + pv
                    return rcarry

                lax.fori_loop(0, CH, add_row, 0)
                for bat in range(batch):
                    store_cp(p, bat, outb, ss).start()

                @pl.when(p + 2 < npb)
                def _():
                    gather_cp(p + 2, inb, gs).start()
                    pe_cp(p + 2, peb, ps).start()
            return carry

        lax.fori_loop(0, npb // 2, iter2, 0)

        for par in range(2):
            inb, outb, peb, gs, ps, ss = bufs[par]
            for bat in range(batch):
                store_cp(npb - 2 + par, bat, outb, ss).wait()

    return emb_kernel


def kernel(token_ids, table):
    b, s = token_ids.shape
    vocab, d = table.shape
    ids = token_ids.astype(jnp.int32).reshape(b, NW, -1, CH).transpose(1, 2, 0, 3)
    ids = ids.reshape(NW, -1, b * CH)
    pe = jnp.asarray(_pe_table_np(s, d))
    out = _build(b, s, vocab, d)(ids, table, pe)
    return out.reshape(b, s, d)
